# baseline (speedup 1.0000x reference)
import jax
import jax.numpy as jnp
from jax import lax
from jax.experimental import pallas as pl
from jax.experimental.pallas import tpu as pltpu, tpu_sc as plsc

N = 10000
E = 320000
D = 128
H = 8
DH = 16
SCALE = 0.25

NC = 2
NS = 16
NW = NC * NS
C = 64
NCHUNKS = E // C
NPAD = 10240
ROWS_PER_SUB = NPAD // NS


def _proj_body(x_ref, s_ref, wq_ref, bq_ref, wk_ref, bk_ref, wv_ref, bv_ref,
               q_ref, kv_ref):
    x = x_ref[...]
    xs = x + s_ref[...]
    dn = (((1,), (1,)), ((), ()))
    q = lax.dot_general(xs, wq_ref[...], dn, precision=lax.Precision.HIGHEST,
                        preferred_element_type=jnp.float32)
    k = lax.dot_general(xs, wk_ref[...], dn, precision=lax.Precision.HIGHEST,
                        preferred_element_type=jnp.float32)
    v = lax.dot_general(x, wv_ref[...], dn, precision=lax.Precision.HIGHEST,
                        preferred_element_type=jnp.float32)
    q_ref[...] = (q + bq_ref[...]) * SCALE
    kv_ref[:, :D] = k + bk_ref[...]
    kv_ref[:, D:] = v + bv_ref[...]


def _proj(in_feats, struc_feats, Wq, bq, Wk, bk, Wv, bv):
    BN = 1000
    grid = (N // BN,)
    row_spec = pl.BlockSpec((BN, D), lambda i: (i, 0))
    w_spec = pl.BlockSpec((D, D), lambda i: (0, 0))
    b_spec = pl.BlockSpec((1, D), lambda i: (0, 0))
    return pl.pallas_call(
        _proj_body,
        grid=grid,
        in_specs=[row_spec, row_spec, w_spec, b_spec, w_spec, b_spec,
                  w_spec, b_spec],
        out_specs=[row_spec, pl.BlockSpec((BN, 2 * D), lambda i: (i, 0))],
        out_shape=[jax.ShapeDtypeStruct((N, D), jnp.float32),
                   jax.ShapeDtypeStruct((N, 2 * D), jnp.float32)],
    )(in_feats, struc_feats, Wq, bq.reshape(1, D), Wk, bk.reshape(1, D),
      Wv, bv.reshape(1, D))


def _edge_body(q_hbm, kv_hbm, eidx_hbm, out_hbm,
               gi2, sidx, didx, qg, kvg, msg, agg, sem_q, sem_kv):
    cid = lax.axis_index("c")
    sid = lax.axis_index("s")
    wid = sid * NC + cid

    def zrow(i, _):
        for j in range(D // 16):
            msg[i, pl.ds(j * 16, 16)] = jnp.zeros((16,), jnp.float32)
        return 0
    lax.fori_loop(0, C, zrow, 0)
    for t in range(ROWS_PER_SUB // C):
        pltpu.sync_copy(msg,
                        agg.at[pl.ds(sid * ROWS_PER_SUB + t * C, C)])
    plsc.subcore_barrier()

    rem = NCHUNKS - (NCHUNKS // NW) * NW
    trips = jnp.where(wid < rem, NCHUNKS // NW + 1, NCHUNKS // NW)

    def chunk(t, _):
        g = wid + NW * t
        pltpu.sync_copy(eidx_hbm.at[pl.ds(g * 2 * C, 2 * C)], gi2)
        for j in range(C // 16):
            sidx[pl.ds(j * 16, 16)] = gi2[pl.ds(j * 16, 16)]
            didx[pl.ds(j * 16, 16)] = gi2[pl.ds(C + j * 16, 16)]
        cp_q = pltpu.async_copy(q_hbm.at[sidx], qg, sem_q)
        cp_kv = pltpu.async_copy(kv_hbm.at[didx], kvg, sem_kv)
        cp_q.wait()
        cp_kv.wait()

        @plsc.parallel_loop(0, C, step=1, unroll=8)
        def edge(e):
            for h in range(H):
                qv = qg[e, pl.ds(h * DH, DH)]
                kv = kvg[e, pl.ds(h * DH, DH)]
                vv = kvg[e, pl.ds(D + h * DH, DH)]
                s = jnp.sum(qv * kv)
                w = 1.0 / (1.0 + jnp.exp(jnp.full((16,), -s, jnp.float32)))
                msg[e, pl.ds(h * DH, DH)] = w * vv

        pltpu.sync_copy(msg, agg.at[sidx], add=True)
        return 0
    lax.fori_loop(0, trips, chunk, 0)

    plsc.subcore_barrier()
    pltpu.sync_copy(agg.at[pl.ds(sid * ROWS_PER_SUB, ROWS_PER_SUB)],
                    out_hbm.at[cid, pl.ds(sid * ROWS_PER_SUB, ROWS_PER_SUB)])


def _edge_sc(q, kv, eidx):
    mesh = plsc.VectorSubcoreMesh(core_axis_name="c", subcore_axis_name="s")
    kern = pl.kernel(
        _edge_body,
        out_type=jax.ShapeDtypeStruct((NC, NPAD, D), jnp.float32),
        mesh=mesh,
        compiler_params=pltpu.CompilerParams(needs_layout_passes=False),
        scratch_types=[
            pltpu.VMEM((2 * C,), jnp.int32),
            pltpu.VMEM((C,), jnp.int32),
            pltpu.VMEM((C,), jnp.int32),
            pltpu.VMEM((C, D), jnp.float32),
            pltpu.VMEM((C, 2 * D), jnp.float32),
            pltpu.VMEM((C, D), jnp.float32),
            pltpu.VMEM_SHARED((NPAD, D), jnp.float32),
            pltpu.SemaphoreType.DMA,
            pltpu.SemaphoreType.DMA,
        ],
    )
    return kern(q, kv, eidx)


def _final_body(x_ref, p_ref, wo_ref, o_ref):
    s = p_ref[0] + p_ref[1]
    dn = (((1,), (1,)), ((), ()))
    o_ref[...] = x_ref[...] + lax.dot_general(
        s, wo_ref[...], dn, precision=lax.Precision.HIGHEST,
        preferred_element_type=jnp.float32)


def _final(in_feats, partials, Wo):
    BN = 1000
    grid = (N // BN,)
    row_spec = pl.BlockSpec((BN, D), lambda i: (i, 0))
    return pl.pallas_call(
        _final_body,
        grid=grid,
        in_specs=[row_spec,
                  pl.BlockSpec((NC, BN, D), lambda i: (0, i, 0)),
                  pl.BlockSpec((D, D), lambda i: (0, 0))],
        out_specs=row_spec,
        out_shape=jax.ShapeDtypeStruct((N, D), jnp.float32),
    )(in_feats, partials, Wo)


def kernel(in_feats, edge_ids, struc_feats, Wq, bq, Wk, bk, Wv, bv, Wo):
    q, kv = _proj(in_feats, struc_feats, Wq, bq, Wk, bk, Wv, bv)
    eidx = jnp.transpose(edge_ids.reshape(2, NCHUNKS, C),
                         (1, 0, 2)).reshape(-1)
    partials = _edge_sc(q, kv, eidx)
    return _final(in_feats, partials, Wo)


# speedup vs baseline: 3.5322x; 3.5322x over previous
import jax
import jax.numpy as jnp
from jax import lax
from jax.experimental import pallas as pl
from jax.experimental.pallas import tpu as pltpu, tpu_sc as plsc

N = 10000
E = 320000
D = 128
H = 8
DH = 16
SCALE = 0.25

NC = 2
NS = 16
NW = NC * NS
C = 64
NCHUNKS = E // C
NPAD = 10240
ROWS_PER_SUB = NPAD // NS


def _proj_body(x_ref, s_ref, wq_ref, bq_ref, wk_ref, bk_ref, wv_ref, bv_ref,
               q_ref, kv_ref):
    x = x_ref[...]
    xs = x + s_ref[...]
    dn = (((1,), (1,)), ((), ()))
    q = lax.dot_general(xs, wq_ref[...], dn, precision=lax.Precision.HIGHEST,
                        preferred_element_type=jnp.float32)
    k = lax.dot_general(xs, wk_ref[...], dn, precision=lax.Precision.HIGHEST,
                        preferred_element_type=jnp.float32)
    v = lax.dot_general(x, wv_ref[...], dn, precision=lax.Precision.HIGHEST,
                        preferred_element_type=jnp.float32)
    q_ref[...] = (q + bq_ref[...]) * SCALE
    kv_ref[:, :D] = k + bk_ref[...]
    kv_ref[:, D:] = v + bv_ref[...]


def _proj(in_feats, struc_feats, Wq, bq, Wk, bk, Wv, bv):
    BN = 1000
    grid = (N // BN,)
    row_spec = pl.BlockSpec((BN, D), lambda i: (i, 0))
    w_spec = pl.BlockSpec((D, D), lambda i: (0, 0))
    b_spec = pl.BlockSpec((1, D), lambda i: (0, 0))
    return pl.pallas_call(
        _proj_body,
        grid=grid,
        in_specs=[row_spec, row_spec, w_spec, b_spec, w_spec, b_spec,
                  w_spec, b_spec],
        out_specs=[row_spec, pl.BlockSpec((BN, 2 * D), lambda i: (i, 0))],
        out_shape=[jax.ShapeDtypeStruct((N, D), jnp.float32),
                   jax.ShapeDtypeStruct((N, 2 * D), jnp.float32)],
    )(in_feats, struc_feats, Wq, bq.reshape(1, D), Wk, bk.reshape(1, D),
      Wv, bv.reshape(1, D))


def _edge_body(q_hbm, kv_hbm, eidx_hbm, out_hbm,
               gi2, sidx, didx, qg, kvg, msg, agg, sem_q, sem_kv):
    cid = lax.axis_index("c")
    sid = lax.axis_index("s")
    wid = sid * NC + cid

    def zrow(i, _):
        for j in range(D // 16):
            msg[i, pl.ds(j * 16, 16)] = jnp.zeros((16,), jnp.float32)
        return 0
    lax.fori_loop(0, C, zrow, 0)
    for t in range(ROWS_PER_SUB // C):
        pltpu.sync_copy(msg,
                        agg.at[pl.ds(sid * ROWS_PER_SUB + t * C, C)])
    plsc.subcore_barrier()

    rem = NCHUNKS - (NCHUNKS // NW) * NW
    trips = jnp.where(wid < rem, NCHUNKS // NW + 1, NCHUNKS // NW)

    def chunk(t, _):
        g = wid + NW * t
        pltpu.sync_copy(eidx_hbm.at[pl.ds(g * 2 * C, 2 * C)], gi2)
        for j in range(C // 16):
            sidx[pl.ds(j * 16, 16)] = gi2[pl.ds(j * 16, 16)]
            didx[pl.ds(j * 16, 16)] = gi2[pl.ds(C + j * 16, 16)]
        cp_q = pltpu.async_copy(q_hbm.at[sidx], qg, sem_q)
        cp_kv = pltpu.async_copy(kv_hbm.at[didx], kvg, sem_kv)
        cp_q.wait()
        cp_kv.wait()

        @plsc.parallel_loop(0, C, step=1, unroll=4)
        def edge(e):
            for h in range(H):
                qv = qg[e, pl.ds(h * DH, DH)]
                kv = kvg[e, pl.ds(h * DH, DH)]
                vv = kvg[e, pl.ds(D + h * DH, DH)]
                s = jnp.sum(qv * kv)
                w = 1.0 / (1.0 + jnp.exp(jnp.full((16,), -s, jnp.float32)))
                msg[e, pl.ds(h * DH, DH)] = w * vv

        pltpu.sync_copy(msg, agg.at[sidx], add=True)
        return 0
    lax.fori_loop(0, trips, chunk, 0)

    plsc.subcore_barrier()
    pltpu.sync_copy(agg.at[pl.ds(sid * ROWS_PER_SUB, ROWS_PER_SUB)],
                    out_hbm.at[cid, pl.ds(sid * ROWS_PER_SUB, ROWS_PER_SUB)])


def _edge_sc(q, kv, eidx):
    mesh = plsc.VectorSubcoreMesh(core_axis_name="c", subcore_axis_name="s")
    kern = pl.kernel(
        _edge_body,
        out_type=jax.ShapeDtypeStruct((NC, NPAD, D), jnp.float32),
        mesh=mesh,
        compiler_params=pltpu.CompilerParams(needs_layout_passes=False),
        scratch_types=[
            pltpu.VMEM((2 * C,), jnp.int32),
            pltpu.VMEM((C,), jnp.int32),
            pltpu.VMEM((C,), jnp.int32),
            pltpu.VMEM((C, D), jnp.float32),
            pltpu.VMEM((C, 2 * D), jnp.float32),
            pltpu.VMEM((C, D), jnp.float32),
            pltpu.VMEM_SHARED((NPAD, D), jnp.float32),
            pltpu.SemaphoreType.DMA,
            pltpu.SemaphoreType.DMA,
        ],
    )
    return kern(q, kv, eidx)


def _final_body(x_ref, p_ref, wo_ref, o_ref):
    s = p_ref[0] + p_ref[1]
    dn = (((1,), (1,)), ((), ()))
    o_ref[...] = x_ref[...] + lax.dot_general(
        s, wo_ref[...], dn, precision=lax.Precision.HIGHEST,
        preferred_element_type=jnp.float32)


def _final(in_feats, partials, Wo):
    BN = 1000
    grid = (N // BN,)
    row_spec = pl.BlockSpec((BN, D), lambda i: (i, 0))
    return pl.pallas_call(
        _final_body,
        grid=grid,
        in_specs=[row_spec,
                  pl.BlockSpec((NC, BN, D), lambda i: (0, i, 0)),
                  pl.BlockSpec((D, D), lambda i: (0, 0))],
        out_specs=row_spec,
        out_shape=jax.ShapeDtypeStruct((N, D), jnp.float32),
    )(in_feats, partials, Wo)


def kernel(in_feats, edge_ids, struc_feats, Wq, bq, Wk, bk, Wv, bv, Wo):
    q, kv = _proj(in_feats, struc_feats, Wq, bq, Wk, bk, Wv, bv)
    eidx = jnp.transpose(edge_ids.reshape(2, NCHUNKS, C),
                         (1, 0, 2)).reshape(-1)
    partials = _edge_sc(q, kv, eidx)
    return _final(in_feats, partials, Wo)


# speedup vs baseline: 4.4892x; 1.2710x over previous
import jax
import jax.numpy as jnp
from jax import lax
from jax.experimental import pallas as pl
from jax.experimental.pallas import tpu as pltpu, tpu_sc as plsc

N = 10000
E = 320000
D = 128
H = 8
DH = 16
SCALE = -0.25

NC = 2
NS = 16
NW = NC * NS
C = 64
NCHUNKS = E // C
NPAD = 10240
ROWS_PER_SUB = NPAD // NS


def _proj_body(x_ref, s_ref, wq_ref, bq_ref, wk_ref, bk_ref, wv_ref, bv_ref,
               q_ref, kv_ref):
    x = x_ref[...]
    xs = x + s_ref[...]
    dn = (((1,), (1,)), ((), ()))
    q = lax.dot_general(xs, wq_ref[...], dn, precision=lax.Precision.HIGHEST,
                        preferred_element_type=jnp.float32)
    k = lax.dot_general(xs, wk_ref[...], dn, precision=lax.Precision.HIGHEST,
                        preferred_element_type=jnp.float32)
    v = lax.dot_general(x, wv_ref[...], dn, precision=lax.Precision.HIGHEST,
                        preferred_element_type=jnp.float32)
    q_ref[...] = (q + bq_ref[...]) * SCALE
    kv_ref[:, :D] = k + bk_ref[...]
    kv_ref[:, D:] = v + bv_ref[...]


def _proj(in_feats, struc_feats, Wq, bq, Wk, bk, Wv, bv):
    BN = 1000
    grid = (N // BN,)
    row_spec = pl.BlockSpec((BN, D), lambda i: (i, 0))
    w_spec = pl.BlockSpec((D, D), lambda i: (0, 0))
    b_spec = pl.BlockSpec((1, D), lambda i: (0, 0))
    return pl.pallas_call(
        _proj_body,
        grid=grid,
        in_specs=[row_spec, row_spec, w_spec, b_spec, w_spec, b_spec,
                  w_spec, b_spec],
        out_specs=[row_spec, pl.BlockSpec((BN, 2 * D), lambda i: (i, 0))],
        out_shape=[jax.ShapeDtypeStruct((N, D), jnp.float32),
                   jax.ShapeDtypeStruct((N, 2 * D), jnp.float32)],
    )(in_feats, struc_feats, Wq, bq.reshape(1, D), Wk, bk.reshape(1, D),
      Wv, bv.reshape(1, D))


def _edge_body(q_hbm, kv_hbm, eidx_hbm, out_hbm,
               gi2, sidx, didx, qg, kvg, msg, agg, sem_q, sem_kv):
    cid = lax.axis_index("c")
    sid = lax.axis_index("s")
    wid = sid * NC + cid

    def zrow(i, _):
        for j in range(D // 16):
            msg[i, pl.ds(j * 16, 16)] = jnp.zeros((16,), jnp.float32)
        return 0
    lax.fori_loop(0, C, zrow, 0)
    for t in range(ROWS_PER_SUB // C):
        pltpu.sync_copy(msg,
                        agg.at[pl.ds(sid * ROWS_PER_SUB + t * C, C)])
    plsc.subcore_barrier()

    rem = NCHUNKS - (NCHUNKS // NW) * NW
    trips = jnp.where(wid < rem, NCHUNKS // NW + 1, NCHUNKS // NW)

    def chunk(t, _):
        g = wid + NW * t
        pltpu.sync_copy(eidx_hbm.at[pl.ds(g * 2 * C, 2 * C)], gi2)
        for j in range(C // 16):
            sidx[pl.ds(j * 16, 16)] = gi2[pl.ds(j * 16, 16)]
            didx[pl.ds(j * 16, 16)] = gi2[pl.ds(C + j * 16, 16)]
        cp_q = pltpu.async_copy(q_hbm.at[sidx], qg, sem_q)
        cp_kv = pltpu.async_copy(kv_hbm.at[didx], kvg, sem_kv)
        cp_q.wait()
        cp_kv.wait()

        @plsc.parallel_loop(0, C, step=1, unroll=4)
        def edge(e):
            for h in range(H):
                qv = qg[e, pl.ds(h * DH, DH)]
                kv = kvg[e, pl.ds(h * DH, DH)]
                vv = kvg[e, pl.ds(D + h * DH, DH)]
                s = jnp.sum(qv * kv)
                den = 1.0 + jnp.exp(jnp.full((16,), s, jnp.float32))
                msg[e, pl.ds(h * DH, DH)] = vv / den

        pltpu.sync_copy(msg, agg.at[sidx], add=True)
        return 0
    lax.fori_loop(0, trips, chunk, 0)

    plsc.subcore_barrier()
    pltpu.sync_copy(agg.at[pl.ds(sid * ROWS_PER_SUB, ROWS_PER_SUB)],
                    out_hbm.at[cid, pl.ds(sid * ROWS_PER_SUB, ROWS_PER_SUB)])


def _edge_sc(q, kv, eidx):
    mesh = plsc.VectorSubcoreMesh(core_axis_name="c", subcore_axis_name="s")
    kern = pl.kernel(
        _edge_body,
        out_type=jax.ShapeDtypeStruct((NC, NPAD, D), jnp.float32),
        mesh=mesh,
        compiler_params=pltpu.CompilerParams(needs_layout_passes=False),
        scratch_types=[
            pltpu.VMEM((2 * C,), jnp.int32),
            pltpu.VMEM((C,), jnp.int32),
            pltpu.VMEM((C,), jnp.int32),
            pltpu.VMEM((C, D), jnp.float32),
            pltpu.VMEM((C, 2 * D), jnp.float32),
            pltpu.VMEM((C, D), jnp.float32),
            pltpu.VMEM_SHARED((NPAD, D), jnp.float32),
            pltpu.SemaphoreType.DMA,
            pltpu.SemaphoreType.DMA,
        ],
    )
    return kern(q, kv, eidx)


def _final_body(x_ref, p_ref, wo_ref, o_ref):
    s = p_ref[0] + p_ref[1]
    dn = (((1,), (1,)), ((), ()))
    o_ref[...] = x_ref[...] + lax.dot_general(
        s, wo_ref[...], dn, precision=lax.Precision.HIGHEST,
        preferred_element_type=jnp.float32)


def _final(in_feats, partials, Wo):
    BN = 1000
    grid = (N // BN,)
    row_spec = pl.BlockSpec((BN, D), lambda i: (i, 0))
    return pl.pallas_call(
        _final_body,
        grid=grid,
        in_specs=[row_spec,
                  pl.BlockSpec((NC, BN, D), lambda i: (0, i, 0)),
                  pl.BlockSpec((D, D), lambda i: (0, 0))],
        out_specs=row_spec,
        out_shape=jax.ShapeDtypeStruct((N, D), jnp.float32),
    )(in_feats, partials, Wo)


def kernel(in_feats, edge_ids, struc_feats, Wq, bq, Wk, bk, Wv, bv, Wo):
    q, kv = _proj(in_feats, struc_feats, Wq, bq, Wk, bk, Wv, bv)
    eidx = jnp.transpose(edge_ids.reshape(2, NCHUNKS, C),
                         (1, 0, 2)).reshape(-1)
    partials = _edge_sc(q, kv, eidx)
    return _final(in_feats, partials, Wo)
